# baseline (speedup 1.0000x reference)
import functools

import jax
import jax.numpy as jnp
from jax import lax
from jax.experimental import pallas as pl
from jax.experimental.pallas import tpu as pltpu
from jax.experimental.pallas import tpu_sc as plsc

from jax._src.pallas.mosaic import core as _tpu_core
from jax._src.pallas.mosaic import lowering as _mosaic_lowering

_mosaic_lowering.lowering_rules[_tpu_core.CoreType.SC_VECTOR_SUBCORE][
    lax.tanh_p
] = _mosaic_lowering._tanh_lowering_rule

PI = 3.141592653589793
GAMMA = 12.0
HIDDEN_DIM = 64
EMBEDDING_RANGE = (GAMMA + 2.0) / HIDDEN_DIM

C_AX = PI / EMBEDDING_RANGE
C_AR = 2.0 * PI / EMBEDDING_RANGE

NW = 32
CB = 4
L = 20
CHUNK = CB * L
IDXW = 80


def _body(
    table_hbm, idx_hbm, ax_hbm, ar_hbm,
    idx_v, rows0, rows1, ax0, ar0, ax1, ar1,
    sem_g0, sem_g1, sem_o0, sem_o1,
):
    wid = lax.axis_index("s") * 2 + lax.axis_index("c")
    bpw = ax_hbm.shape[0] // NW
    n_chunks = bpw // CB
    n2 = n_chunks // 2
    idx_rows = idx_v.shape[0]

    pltpu.sync_copy(idx_hbm.at[pl.ds(wid * idx_rows, idx_rows)], idx_v)

    def start_gather(c, rows, sem):
        pltpu.async_copy(table_hbm.at[idx_v.at[c]], rows, sem)

    def drain_gather(rows, sem):
        pltpu.make_async_copy(table_hbm.at[pl.ds(0, CHUNK)], rows, sem).wait()

    def drain_out(ax_b, ar_b, sem):
        pltpu.make_async_copy(ax_hbm.at[pl.ds(0, CB)], ax_b, sem).wait()
        pltpu.make_async_copy(ar_hbm.at[pl.ds(0, CB)], ar_b, sem).wait()

    def compute(rows, ax_b, ar_b):
        @plsc.parallel_loop(0, CHUNK, step=1, unroll=8)
        def row_step(r):
            b = r // L
            t = r - b * L
            for j in range(4):
                x = rows[r, pl.ds(j * 16, 16)]
                ax_b[b, t, pl.ds(j * 16, 16)] = PI * jnp.tanh(x * C_AX)
            for j in range(4):
                x = rows[r, pl.ds(64 + j * 16, 16)]
                u = jnp.tanh(x * C_AR)
                ar_b[b, t, pl.ds(j * 16, 16)] = (PI / 2.0) * u + (PI / 2.0)

    def start_out(c, ax_b, ar_b, sem):
        base = wid * bpw + c * CB
        pltpu.async_copy(ax_b, ax_hbm.at[pl.ds(base, CB)], sem)
        pltpu.async_copy(ar_b, ar_hbm.at[pl.ds(base, CB)], sem)

    start_gather(0, rows0, sem_g0)

    def step(i, carry):
        c0 = 2 * i
        c1 = c0 + 1
        start_gather(c1, rows1, sem_g1)
        drain_gather(rows0, sem_g0)

        @pl.when(i > 0)
        def _():
            drain_out(ax0, ar0, sem_o0)

        compute(rows0, ax0, ar0)
        start_out(c0, ax0, ar0, sem_o0)

        @pl.when(i + 1 < n2)
        def _():
            start_gather(c0 + 2, rows0, sem_g0)

        drain_gather(rows1, sem_g1)

        @pl.when(i > 0)
        def _():
            drain_out(ax1, ar1, sem_o1)

        compute(rows1, ax1, ar1)
        start_out(c1, ax1, ar1, sem_o1)
        return carry

    lax.fori_loop(0, n2, step, 0)
    drain_out(ax0, ar0, sem_o0)
    drain_out(ax1, ar1, sem_o1)


def kernel(entity_embedding, indices):
    b, l = indices.shape
    n = b * l
    assert l == L and b % (NW * CB) == 0 and n % IDXW == 0
    idx2d = indices.reshape(n // IDXW, IDXW)
    idx_rows_per_w = idx2d.shape[0] // NW

    mesh = plsc.VectorSubcoreMesh(core_axis_name="c", subcore_axis_name="s")
    run = functools.partial(
        pl.kernel,
        out_type=[
            jax.ShapeDtypeStruct((b, L, HIDDEN_DIM), jnp.float32),
            jax.ShapeDtypeStruct((b, L, HIDDEN_DIM), jnp.float32),
        ],
        mesh=mesh,
        scratch_types=[
            pltpu.VMEM((idx_rows_per_w, IDXW), jnp.int32),
            pltpu.VMEM((CHUNK, 2 * HIDDEN_DIM), jnp.float32),
            pltpu.VMEM((CHUNK, 2 * HIDDEN_DIM), jnp.float32),
            pltpu.VMEM((CB, L, HIDDEN_DIM), jnp.float32),
            pltpu.VMEM((CB, L, HIDDEN_DIM), jnp.float32),
            pltpu.VMEM((CB, L, HIDDEN_DIM), jnp.float32),
            pltpu.VMEM((CB, L, HIDDEN_DIM), jnp.float32),
            pltpu.SemaphoreType.DMA,
            pltpu.SemaphoreType.DMA,
            pltpu.SemaphoreType.DMA,
            pltpu.SemaphoreType.DMA,
        ],
    )(_body)

    ax, ar = run(entity_embedding, idx2d)
    return ax, ar


# speedup vs baseline: 1.0484x; 1.0484x over previous
import functools

import jax
import jax.numpy as jnp
from jax import lax
from jax.experimental import pallas as pl
from jax.experimental.pallas import tpu as pltpu
from jax.experimental.pallas import tpu_sc as plsc

from jax._src.pallas.mosaic import core as _tpu_core
from jax._src.pallas.mosaic import lowering as _mosaic_lowering

_mosaic_lowering.lowering_rules[_tpu_core.CoreType.SC_VECTOR_SUBCORE][
    lax.tanh_p
] = _mosaic_lowering._tanh_lowering_rule

PI = 3.141592653589793
GAMMA = 12.0
HIDDEN_DIM = 64
EMBEDDING_RANGE = (GAMMA + 2.0) / HIDDEN_DIM

C_AX = PI / EMBEDDING_RANGE
C_AR = 2.0 * PI / EMBEDDING_RANGE

NW = 32
CB = 4
L = 20
CHUNK = CB * L
IDXW = 80


def _body(
    table_hbm, idx_hbm, ax_hbm, ar_hbm,
    idx_v, rows0, rows1, ax0, ar0, ax1, ar1,
    sem_g0, sem_g1, sem_o0, sem_o1,
):
    wid = lax.axis_index("s") * 2 + lax.axis_index("c")
    bpw = ax_hbm.shape[0] // NW
    n_chunks = bpw // CB
    n2 = n_chunks // 2
    idx_rows = idx_v.shape[0]

    pltpu.sync_copy(idx_hbm.at[pl.ds(wid * idx_rows, idx_rows)], idx_v)

    def start_gather(c, rows, sem):
        pltpu.async_copy(table_hbm.at[idx_v.at[c]], rows, sem)

    def drain_gather(rows, sem):
        pltpu.make_async_copy(table_hbm.at[pl.ds(0, CHUNK)], rows, sem).wait()

    def drain_out(ax_b, ar_b, sem):
        pltpu.make_async_copy(ax_hbm.at[pl.ds(0, CB)], ax_b, sem).wait()
        pltpu.make_async_copy(ar_hbm.at[pl.ds(0, CB)], ar_b, sem).wait()

    def compute(rows, ax_b, ar_b):
        @plsc.parallel_loop(0, CHUNK, step=1, unroll=4)
        def row_step(r):
            b = r // L
            t = r - b * L
            for j in range(4):
                x = rows[r, pl.ds(j * 16, 16)]
                ax_b[b, t, pl.ds(j * 16, 16)] = PI * jnp.tanh(x * C_AX)
            for j in range(4):
                x = rows[r, pl.ds(64 + j * 16, 16)]
                u = jnp.tanh(x * C_AR)
                ar_b[b, t, pl.ds(j * 16, 16)] = (PI / 2.0) * u + (PI / 2.0)

    def start_out(c, ax_b, ar_b, sem):
        base = wid * bpw + c * CB
        pltpu.async_copy(ax_b, ax_hbm.at[pl.ds(base, CB)], sem)
        pltpu.async_copy(ar_b, ar_hbm.at[pl.ds(base, CB)], sem)

    start_gather(0, rows0, sem_g0)

    def step(i, carry):
        c0 = 2 * i
        c1 = c0 + 1
        start_gather(c1, rows1, sem_g1)
        drain_gather(rows0, sem_g0)

        @pl.when(i > 0)
        def _():
            drain_out(ax0, ar0, sem_o0)

        compute(rows0, ax0, ar0)
        start_out(c0, ax0, ar0, sem_o0)

        @pl.when(i + 1 < n2)
        def _():
            start_gather(c0 + 2, rows0, sem_g0)

        drain_gather(rows1, sem_g1)

        @pl.when(i > 0)
        def _():
            drain_out(ax1, ar1, sem_o1)

        compute(rows1, ax1, ar1)
        start_out(c1, ax1, ar1, sem_o1)
        return carry

    lax.fori_loop(0, n2, step, 0)
    drain_out(ax0, ar0, sem_o0)
    drain_out(ax1, ar1, sem_o1)


def kernel(entity_embedding, indices):
    b, l = indices.shape
    n = b * l
    assert l == L and b % (NW * CB) == 0 and n % IDXW == 0
    idx2d = indices.reshape(n // IDXW, IDXW)
    idx_rows_per_w = idx2d.shape[0] // NW

    mesh = plsc.VectorSubcoreMesh(core_axis_name="c", subcore_axis_name="s")
    run = functools.partial(
        pl.kernel,
        out_type=[
            jax.ShapeDtypeStruct((b, L, HIDDEN_DIM), jnp.float32),
            jax.ShapeDtypeStruct((b, L, HIDDEN_DIM), jnp.float32),
        ],
        mesh=mesh,
        scratch_types=[
            pltpu.VMEM((idx_rows_per_w, IDXW), jnp.int32),
            pltpu.VMEM((CHUNK, 2 * HIDDEN_DIM), jnp.float32),
            pltpu.VMEM((CHUNK, 2 * HIDDEN_DIM), jnp.float32),
            pltpu.VMEM((CB, L, HIDDEN_DIM), jnp.float32),
            pltpu.VMEM((CB, L, HIDDEN_DIM), jnp.float32),
            pltpu.VMEM((CB, L, HIDDEN_DIM), jnp.float32),
            pltpu.VMEM((CB, L, HIDDEN_DIM), jnp.float32),
            pltpu.SemaphoreType.DMA,
            pltpu.SemaphoreType.DMA,
            pltpu.SemaphoreType.DMA,
            pltpu.SemaphoreType.DMA,
        ],
    )(_body)

    ax, ar = run(entity_embedding, idx2d)
    return ax, ar
